# baseline (speedup 1.0000x reference)
import functools

import jax
import jax.numpy as jnp
from jax import lax
from jax.experimental import pallas as pl
from jax.experimental.pallas import tpu as pltpu
from jax.experimental.pallas import tpu_sc as plsc

BATCH = 16384
DIM = 128
NROWS = 13
NPAIR = NROWS * NROWS
NREP = 8
NC = 2
NS = 16
NW = NC * NS
PAIRS = BATCH // 2
P_PER_W = PAIRS // NW
CHUNK = 128
NCHUNK = P_PER_W // CHUNK


def _embed_body(table_hbm, pidx_hbm, out_hbm, idx_v, rows_v, *sems):
    gsem = sems[:NCHUNK]
    wid = lax.axis_index("s") * NC + lax.axis_index("c")
    base = wid * P_PER_W
    pltpu.sync_copy(pidx_hbm.at[wid], idx_v)
    gops = [
        pltpu.async_copy(
            table_hbm.at[idx_v.at[j]], rows_v.at[pl.ds(j * CHUNK, CHUNK)],
            gsem[j],
        )
        for j in range(NCHUNK)
    ]
    for op in gops:
        op.wait()
    pltpu.sync_copy(rows_v, out_hbm.at[pl.ds(base, P_PER_W)])


_embed = functools.partial(
    pl.kernel,
    out_type=jax.ShapeDtypeStruct((PAIRS, 2 * DIM), jnp.float32),
    scratch_types=(
        [pltpu.VMEM((NCHUNK, CHUNK), jnp.int32)]
        + [pltpu.VMEM((P_PER_W, 2 * DIM), jnp.float32)]
        + [pltpu.SemaphoreType.DMA for _ in range(NCHUNK)]
    ),
    mesh=plsc.VectorSubcoreMesh(core_axis_name="c", subcore_axis_name="s"),
)(_embed_body)


def kernel(month, table):
    m = month
    if m.ndim == 2:
        m = jnp.squeeze(m, axis=-1)
    t = table.astype(jnp.float32)
    pair = jnp.concatenate(
        [jnp.repeat(t, NROWS, axis=0), jnp.tile(t, (NROWS, 1))], axis=1
    )
    pair_rep = jnp.tile(pair, (NW, 1))
    m2 = m.astype(jnp.int32).reshape(PAIRS, 2)
    pidx = m2[:, 0] * NROWS + m2[:, 1]
    pidx = pidx.reshape(NW, NCHUNK, CHUNK)
    w = jnp.arange(NW, dtype=jnp.int32)
    pidx = pidx + (w * NPAIR)[:, None, None]
    out2 = _embed(pair_rep, pidx)
    return out2.reshape(BATCH, DIM)


# speedup vs baseline: 1.9028x; 1.9028x over previous
import functools

import jax
import jax.numpy as jnp
from jax import lax
from jax.experimental import pallas as pl
from jax.experimental.pallas import tpu as pltpu
from jax.experimental.pallas import tpu_sc as plsc

BATCH = 16384
DIM = 128
NROWS = 13
NC = 2
NS = 16
NW = NC * NS
B_PER_W = BATCH // NW
CHUNK = 128
NCHUNK = B_PER_W // CHUNK


def _embed_body(table_hbm, month_hbm, out_hbm, idx_v, rows_v, *sems):
    gsem = sems[:NCHUNK]
    ssem = sems[NCHUNK:]
    wid = lax.axis_index("s") * NC + lax.axis_index("c")
    base = wid * B_PER_W
    pltpu.sync_copy(month_hbm.at[wid], idx_v)
    gops = [
        pltpu.async_copy(
            table_hbm.at[idx_v.at[j]], rows_v.at[pl.ds(j * CHUNK, CHUNK)],
            gsem[j],
        )
        for j in range(NCHUNK)
    ]
    half = B_PER_W // 2
    gops[0].wait()
    gops[1].wait()
    s0 = pltpu.async_copy(
        rows_v.at[pl.ds(0, half)], out_hbm.at[pl.ds(base, half)], ssem[0]
    )
    gops[2].wait()
    gops[3].wait()
    s1 = pltpu.async_copy(
        rows_v.at[pl.ds(half, half)], out_hbm.at[pl.ds(base + half, half)],
        ssem[1],
    )
    s0.wait()
    s1.wait()


_embed = functools.partial(
    pl.kernel,
    out_type=jax.ShapeDtypeStruct((BATCH, DIM), jnp.float32),
    scratch_types=(
        [pltpu.VMEM((NCHUNK, CHUNK), jnp.int32)]
        + [pltpu.VMEM((B_PER_W, DIM), jnp.float32)]
        + [pltpu.SemaphoreType.DMA for _ in range(NCHUNK + 2)]
    ),
    mesh=plsc.VectorSubcoreMesh(core_axis_name="c", subcore_axis_name="s"),
)(_embed_body)


def kernel(month, table):
    m = month
    if m.ndim == 2:
        m = jnp.squeeze(m, axis=-1)
    idx = m.astype(jnp.int32).reshape(NW, NCHUNK, CHUNK)
    w = jnp.arange(NW, dtype=jnp.int32)
    i = jnp.arange(CHUNK, dtype=jnp.int32)
    rep = w[:, None, None] * 8 + (i % 8)[None, None, :]
    idx = idx + rep * NROWS
    table_rep = jnp.tile(table.astype(jnp.float32), (NW * 8, 1))
    return _embed(table_rep, idx)
